# Initial kernel scaffold; baseline (speedup 1.0000x reference)
import jax
import jax.numpy as jnp
from jax.experimental import pallas as pl


def kernel(queries, memory, w_pred, w_target):
    raise NotImplementedError("write your pallas kernel here")


# speedup vs baseline: 2.4198x; 2.4198x over previous
import functools

import jax
import jax.numpy as jnp
from jax.experimental import pallas as pl
from jax.experimental.pallas import tpu as pltpu

K_NEIGHBORS = 10
KERNEL_EPS = 1e-5
DENOM_CONST = 1e-5
BONUS_LIMIT = 5.0

_INF = float("inf")
_BIGI = 2**30


def _extract_k_smallest(x, k):
    rows, width = x.shape
    lane = jax.lax.broadcasted_iota(jnp.int32, (rows, width), 1)
    mins = []
    for _ in range(k):
        mn = jnp.min(x, axis=1, keepdims=True)
        mins.append(mn)
        cand = jnp.where(x == mn, lane, _BIGI)
        amin = jnp.min(cand, axis=1, keepdims=True)
        x = jnp.where(lane == amin, _INF, x)
    return jnp.concatenate(mins, axis=1)


def _body(q_ref, m_ref, wp_ref, wt_ref, o_ref, topk_ref, *, nb, mb, m_total):
    i = pl.program_id(0)
    q_rows = q_ref.shape[0]

    @pl.when(i == 0)
    def _init():
        topk_ref[:] = jnp.full(topk_ref.shape, _INF, jnp.float32)

    q = q_ref[:]
    m = m_ref[:]
    q2 = jnp.sum(q * q, axis=1, keepdims=True)
    m2 = jnp.sum(m * m, axis=0, keepdims=True)
    qm = jnp.dot(q, m, preferred_element_type=jnp.float32)
    d2 = q2 + m2 - 2.0 * qm
    d2 = jnp.maximum(d2, 0.0)
    col = i * mb + jax.lax.broadcasted_iota(jnp.int32, d2.shape, 1)
    d2 = jnp.where(col < m_total, d2, _INF)

    bm = _extract_k_smallest(d2, K_NEIGHBORS)

    cur = topk_ref[:, :K_NEIGHBORS]
    combined = jnp.concatenate([cur, bm], axis=1)
    new_top = _extract_k_smallest(combined, K_NEIGHBORS)
    pad = jnp.full((q_rows, topk_ref.shape[1] - K_NEIGHBORS), _INF, jnp.float32)
    topk_ref[:] = jnp.concatenate([new_top, pad], axis=1)

    @pl.when(i == nb - 1)
    def _finalize():
        nn_d2 = topk_ref[:, :K_NEIGHBORS]
        running_mean = jnp.mean(nn_d2) + 1e-12
        kern = KERNEL_EPS / (nn_d2 / running_mean + KERNEL_EPS)
        episodic = 1.0 / jnp.sqrt(jnp.sum(kern, axis=1, keepdims=True) + DENOM_CONST)

        diff = jnp.dot(q, wp_ref[:] - wt_ref[:], preferred_element_type=jnp.float32)
        err = jnp.mean(diff * diff, axis=1, keepdims=True)
        err_mean = jnp.mean(err)
        err_std = jnp.sqrt(jnp.mean((err - err_mean) ** 2)) + 1e-8
        bonus = jnp.clip(1.0 + (err - err_mean) / err_std, 1.0, BONUS_LIMIT)
        o_ref[:] = episodic * bonus


def kernel(queries, memory, w_pred, w_target):
    q_n, d = queries.shape
    m_total = memory.shape[0]
    h = w_pred.shape[1]
    mb = 2048
    nb = pl.cdiv(m_total, mb)
    mem_t = memory.T

    out = pl.pallas_call(
        functools.partial(_body, nb=nb, mb=mb, m_total=m_total),
        grid=(nb,),
        in_specs=[
            pl.BlockSpec((q_n, d), lambda i: (0, 0)),
            pl.BlockSpec((d, mb), lambda i: (0, i)),
            pl.BlockSpec((d, h), lambda i: (0, 0)),
            pl.BlockSpec((d, h), lambda i: (0, 0)),
        ],
        out_specs=pl.BlockSpec((q_n, 1), lambda i: (0, 0)),
        out_shape=jax.ShapeDtypeStruct((q_n, 1), jnp.float32),
        scratch_shapes=[pltpu.VMEM((q_n, 128), jnp.float32)],
    )(queries, mem_t, w_pred, w_target)
    return out[:, 0]


# speedup vs baseline: 9.1049x; 3.7627x over previous
import functools

import jax
import jax.numpy as jnp
from jax.experimental import pallas as pl
from jax.experimental.pallas import tpu as pltpu

K_NEIGHBORS = 10
KERNEL_EPS = 1e-5
DENOM_CONST = 1e-5
BONUS_LIMIT = 5.0

_INF = float("inf")
_BIGI = 2**30
_PAD_VALUE = 1.0e6


def _extract_k_smallest(x, k):
    rows, width = x.shape
    lane = jax.lax.broadcasted_iota(jnp.int32, (rows, width), 1)
    mins = []
    for _ in range(k):
        mn = jnp.min(x, axis=1, keepdims=True)
        mins.append(mn)
        cand = jnp.where(x == mn, lane, _BIGI)
        amin = jnp.min(cand, axis=1, keepdims=True)
        x = jnp.where(lane == amin, _INF, x)
    return jnp.concatenate(mins, axis=1)


def _body(q_ref, m_ref, wp_ref, wt_ref, o_ref, cand_ref, *, nb, sub_blocks):
    i = pl.program_id(0)

    @pl.when(i == 0)
    def _init():
        cand_ref[:] = jnp.full(cand_ref.shape, _INF, jnp.float32)

    q = q_ref[:]
    m = m_ref[:]
    hm2 = 0.5 * jnp.sum(m * m, axis=0, keepdims=True)
    qm = jnp.dot(q, m, preferred_element_type=jnp.float32)

    ms = [cand_ref[:, 128 * j:128 * (j + 1)] for j in range(K_NEIGHBORS)]
    for s in range(sub_blocks):
        v = hm2[:, 128 * s:128 * (s + 1)] - qm[:, 128 * s:128 * (s + 1)]
        for j in range(K_NEIGHBORS):
            t = jnp.minimum(ms[j], v)
            if j + 1 < K_NEIGHBORS:
                v = jnp.maximum(ms[j], v)
            ms[j] = t
    for j in range(K_NEIGHBORS):
        cand_ref[:, 128 * j:128 * (j + 1)] = ms[j]

    @pl.when(i == nb - 1)
    def _finalize():
        top_e = _extract_k_smallest(cand_ref[:], K_NEIGHBORS)
        q2 = jnp.sum(q * q, axis=1, keepdims=True)
        nn_d2 = jnp.maximum(2.0 * top_e + q2, 0.0)
        running_mean = jnp.mean(nn_d2) + 1e-12
        kern = KERNEL_EPS / (nn_d2 / running_mean + KERNEL_EPS)
        episodic = 1.0 / jnp.sqrt(jnp.sum(kern, axis=1, keepdims=True) + DENOM_CONST)

        diff = jnp.dot(q, wp_ref[:] - wt_ref[:], preferred_element_type=jnp.float32)
        err = jnp.mean(diff * diff, axis=1, keepdims=True)
        err_mean = jnp.mean(err)
        err_std = jnp.sqrt(jnp.mean((err - err_mean) ** 2)) + 1e-8
        bonus = jnp.clip(1.0 + (err - err_mean) / err_std, 1.0, BONUS_LIMIT)
        o_ref[:] = episodic * bonus


def kernel(queries, memory, w_pred, w_target):
    q_n, d = queries.shape
    m_total = memory.shape[0]
    h = w_pred.shape[1]
    mb = 4096
    nb = pl.cdiv(m_total, mb)
    m_pad = nb * mb
    mem_t = memory.T
    if m_pad != m_total:
        mem_t = jnp.pad(mem_t, ((0, 0), (0, m_pad - m_total)),
                        constant_values=_PAD_VALUE)

    out = pl.pallas_call(
        functools.partial(_body, nb=nb, sub_blocks=mb // 128),
        grid=(nb,),
        in_specs=[
            pl.BlockSpec((q_n, d), lambda i: (0, 0)),
            pl.BlockSpec((d, mb), lambda i: (0, i)),
            pl.BlockSpec((d, h), lambda i: (0, 0)),
            pl.BlockSpec((d, h), lambda i: (0, 0)),
        ],
        out_specs=pl.BlockSpec((q_n, 1), lambda i: (0, 0)),
        out_shape=jax.ShapeDtypeStruct((q_n, 1), jnp.float32),
        scratch_shapes=[pltpu.VMEM((q_n, 128 * K_NEIGHBORS), jnp.float32)],
    )(queries, mem_t, w_pred, w_target)
    return out[:, 0]


# speedup vs baseline: 11.5998x; 1.2740x over previous
import functools

import jax
import jax.numpy as jnp
from jax.experimental import pallas as pl
from jax.experimental.pallas import tpu as pltpu

K_NEIGHBORS = 10
KERNEL_EPS = 1e-5
DENOM_CONST = 1e-5
BONUS_LIMIT = 5.0

_INF = float("inf")
_BIGI = 2**30
_PAD_VALUE = 1.0e6
_LIST = 16
_CHUNK = 16


def _oddeven_merge(lo, hi, r):
    step = r * 2
    if step < hi - lo:
        yield from _oddeven_merge(lo, hi, step)
        yield from _oddeven_merge(lo + r, hi, step)
        for i in range(lo + r, hi - r, step):
            yield (i, i + r)
    else:
        yield (lo, lo + r)


def _oddeven_merge_sort(lo, hi):
    if (hi - lo) >= 1:
        mid = lo + ((hi - lo) // 2)
        yield from _oddeven_merge_sort(lo, mid)
        yield from _oddeven_merge_sort(mid + 1, hi)
        yield from _oddeven_merge(lo, hi, 1)


def _bitonic_cleanup(n):
    pairs = []
    r = n // 2
    while r >= 1:
        for b in range(0, n, 2 * r):
            for i in range(b, b + r):
                pairs.append((i, i + r))
        r //= 2
    return pairs


_SORT_NET = list(_oddeven_merge_sort(0, _CHUNK - 1))
_CLEAN_NET = _bitonic_cleanup(_LIST)


def _extract_k_smallest(x, k):
    rows, width = x.shape
    lane = jax.lax.broadcasted_iota(jnp.int32, (rows, width), 1)
    mins = []
    for _ in range(k):
        mn = jnp.min(x, axis=1, keepdims=True)
        mins.append(mn)
        cand = jnp.where(x == mn, lane, _BIGI)
        amin = jnp.min(cand, axis=1, keepdims=True)
        x = jnp.where(lane == amin, _INF, x)
    return jnp.concatenate(mins, axis=1)


def _body(q_ref, m_ref, wp_ref, wt_ref, o_ref, cand_ref, *, nb, sub_blocks):
    i = pl.program_id(0)

    @pl.when(i == 0)
    def _init():
        cand_ref[:] = jnp.full(cand_ref.shape, _INF, jnp.float32)

    q = q_ref[:]
    m = m_ref[:]
    hm2 = 0.5 * jnp.sum(m * m, axis=0, keepdims=True)
    qm = jnp.dot(q, m, preferred_element_type=jnp.float32)

    ms = [cand_ref[:, 128 * j:128 * (j + 1)] for j in range(_LIST)]
    for c in range(sub_blocks // _CHUNK):
        b = [hm2[:, 128 * s:128 * (s + 1)] - qm[:, 128 * s:128 * (s + 1)]
             for s in range(c * _CHUNK, (c + 1) * _CHUNK)]
        for x, y in _SORT_NET:
            lo = jnp.minimum(b[x], b[y])
            hi = jnp.maximum(b[x], b[y])
            b[x], b[y] = lo, hi
        for j in range(_LIST):
            ms[j] = jnp.minimum(ms[j], b[_CHUNK - 1 - j])
        for x, y in _CLEAN_NET:
            lo = jnp.minimum(ms[x], ms[y])
            hi = jnp.maximum(ms[x], ms[y])
            ms[x], ms[y] = lo, hi
    for j in range(_LIST):
        cand_ref[:, 128 * j:128 * (j + 1)] = ms[j]

    @pl.when(i == nb - 1)
    def _finalize():
        top_e = _extract_k_smallest(cand_ref[:], K_NEIGHBORS)
        q2 = jnp.sum(q * q, axis=1, keepdims=True)
        nn_d2 = jnp.maximum(2.0 * top_e + q2, 0.0)
        running_mean = jnp.mean(nn_d2) + 1e-12
        kern = KERNEL_EPS / (nn_d2 / running_mean + KERNEL_EPS)
        episodic = 1.0 / jnp.sqrt(jnp.sum(kern, axis=1, keepdims=True) + DENOM_CONST)

        diff = jnp.dot(q, wp_ref[:] - wt_ref[:], preferred_element_type=jnp.float32)
        err = jnp.mean(diff * diff, axis=1, keepdims=True)
        err_mean = jnp.mean(err)
        err_std = jnp.sqrt(jnp.mean((err - err_mean) ** 2)) + 1e-8
        bonus = jnp.clip(1.0 + (err - err_mean) / err_std, 1.0, BONUS_LIMIT)
        o_ref[:] = episodic * bonus


def kernel(queries, memory, w_pred, w_target):
    q_n, d = queries.shape
    m_total = memory.shape[0]
    h = w_pred.shape[1]
    mb = 4096
    nb = pl.cdiv(m_total, mb)
    m_pad = nb * mb
    mem_t = memory.T
    if m_pad != m_total:
        mem_t = jnp.pad(mem_t, ((0, 0), (0, m_pad - m_total)),
                        constant_values=_PAD_VALUE)

    out = pl.pallas_call(
        functools.partial(_body, nb=nb, sub_blocks=mb // 128),
        grid=(nb,),
        in_specs=[
            pl.BlockSpec((q_n, d), lambda i: (0, 0)),
            pl.BlockSpec((d, mb), lambda i: (0, i)),
            pl.BlockSpec((d, h), lambda i: (0, 0)),
            pl.BlockSpec((d, h), lambda i: (0, 0)),
        ],
        out_specs=pl.BlockSpec((q_n, 1), lambda i: (0, 0)),
        out_shape=jax.ShapeDtypeStruct((q_n, 1), jnp.float32),
        scratch_shapes=[pltpu.VMEM((q_n, 128 * _LIST), jnp.float32)],
    )(queries, mem_t, w_pred, w_target)
    return out[:, 0]


# speedup vs baseline: 12.4697x; 1.0750x over previous
import functools

import jax
import jax.numpy as jnp
from jax.experimental import pallas as pl
from jax.experimental.pallas import tpu as pltpu

K_NEIGHBORS = 10
KERNEL_EPS = 1e-5
DENOM_CONST = 1e-5
BONUS_LIMIT = 5.0

_INF = float("inf")
_BIGI = 2**30
_PAD_VALUE = 1.0e6
_LIST = 16
_CHUNK = 16


def _oddeven_merge(lo, hi, r):
    step = r * 2
    if step < hi - lo:
        yield from _oddeven_merge(lo, hi, step)
        yield from _oddeven_merge(lo + r, hi, step)
        for i in range(lo + r, hi - r, step):
            yield (i, i + r)
    else:
        yield (lo, lo + r)


def _oddeven_merge_sort(lo, hi):
    if (hi - lo) >= 1:
        mid = lo + ((hi - lo) // 2)
        yield from _oddeven_merge_sort(lo, mid)
        yield from _oddeven_merge_sort(mid + 1, hi)
        yield from _oddeven_merge(lo, hi, 1)


def _bitonic_cleanup(n):
    pairs = []
    r = n // 2
    while r >= 1:
        for b in range(0, n, 2 * r):
            for i in range(b, b + r):
                pairs.append((i, i + r))
        r //= 2
    return pairs


_SORT_NET = list(_oddeven_merge_sort(0, _CHUNK - 1))
_CLEAN_NET = _bitonic_cleanup(_LIST)


def _extract_k_smallest(x, k):
    rows, width = x.shape
    lane = jax.lax.broadcasted_iota(jnp.int32, (rows, width), 1)
    mins = []
    for _ in range(k):
        mn = jnp.min(x, axis=1, keepdims=True)
        mins.append(mn)
        cand = jnp.where(x == mn, lane, _BIGI)
        amin = jnp.min(cand, axis=1, keepdims=True)
        x = jnp.where(lane == amin, _INF, x)
    return jnp.concatenate(mins, axis=1)


def _body(q_ref, m_ref, wp_ref, wt_ref, o_ref, cand_ref, *, nb, sub_blocks):
    i = pl.program_id(0)

    @pl.when(i == 0)
    def _init():
        cand_ref[:] = jnp.full(cand_ref.shape, _INF, jnp.bfloat16)

    q = q_ref[:]
    m = m_ref[:]
    hm2 = 0.5 * jnp.sum(m * m, axis=0, keepdims=True)
    qm = jnp.dot(q, m, preferred_element_type=jnp.float32)

    w = 256
    ms = [cand_ref[:, w * j:w * (j + 1)] for j in range(_LIST)]
    for c in range(sub_blocks // _CHUNK):
        b = [(hm2[:, w * s:w * (s + 1)] - qm[:, w * s:w * (s + 1)]
              ).astype(jnp.bfloat16)
             for s in range(c * _CHUNK, (c + 1) * _CHUNK)]
        for x, y in _SORT_NET:
            lo = jnp.minimum(b[x], b[y])
            hi = jnp.maximum(b[x], b[y])
            b[x], b[y] = lo, hi
        for j in range(_LIST):
            ms[j] = jnp.minimum(ms[j], b[_CHUNK - 1 - j])
        for x, y in _CLEAN_NET:
            lo = jnp.minimum(ms[x], ms[y])
            hi = jnp.maximum(ms[x], ms[y])
            ms[x], ms[y] = lo, hi
    for j in range(_LIST):
        cand_ref[:, w * j:w * (j + 1)] = ms[j]

    @pl.when(i == nb - 1)
    def _finalize():
        cand32 = cand_ref[:].astype(jnp.float32)
        top_e = _extract_k_smallest(cand32, K_NEIGHBORS)
        q2 = jnp.sum(q * q, axis=1, keepdims=True)
        nn_d2 = jnp.maximum(2.0 * top_e + q2, 0.0)
        running_mean = jnp.mean(nn_d2) + 1e-12
        kern = KERNEL_EPS / (nn_d2 / running_mean + KERNEL_EPS)
        episodic = 1.0 / jnp.sqrt(jnp.sum(kern, axis=1, keepdims=True) + DENOM_CONST)

        diff = jnp.dot(q, wp_ref[:] - wt_ref[:], preferred_element_type=jnp.float32)
        err = jnp.mean(diff * diff, axis=1, keepdims=True)
        err_mean = jnp.mean(err)
        err_std = jnp.sqrt(jnp.mean((err - err_mean) ** 2)) + 1e-8
        bonus = jnp.clip(1.0 + (err - err_mean) / err_std, 1.0, BONUS_LIMIT)
        o_ref[:] = episodic * bonus


def kernel(queries, memory, w_pred, w_target):
    q_n, d = queries.shape
    m_total = memory.shape[0]
    h = w_pred.shape[1]
    mb = 4096
    nb = pl.cdiv(m_total, mb)
    m_pad = nb * mb
    mem_t = memory.T
    if m_pad != m_total:
        mem_t = jnp.pad(mem_t, ((0, 0), (0, m_pad - m_total)),
                        constant_values=_PAD_VALUE)

    out = pl.pallas_call(
        functools.partial(_body, nb=nb, sub_blocks=mb // 256),
        grid=(nb,),
        in_specs=[
            pl.BlockSpec((q_n, d), lambda i: (0, 0)),
            pl.BlockSpec((d, mb), lambda i: (0, i)),
            pl.BlockSpec((d, h), lambda i: (0, 0)),
            pl.BlockSpec((d, h), lambda i: (0, 0)),
        ],
        out_specs=pl.BlockSpec((q_n, 1), lambda i: (0, 0)),
        out_shape=jax.ShapeDtypeStruct((q_n, 1), jnp.float32),
        scratch_shapes=[pltpu.VMEM((q_n, 256 * _LIST), jnp.bfloat16)],
    )(queries, mem_t, w_pred, w_target)
    return out[:, 0]


# speedup vs baseline: 12.6212x; 1.0122x over previous
import functools

import jax
import jax.numpy as jnp
from jax.experimental import pallas as pl
from jax.experimental.pallas import tpu as pltpu

K_NEIGHBORS = 10
KERNEL_EPS = 1e-5
DENOM_CONST = 1e-5
BONUS_LIMIT = 5.0

_INF = float("inf")
_BIGI = 2**30
_PAD_VALUE = 1.0e6
_LIST = 16
_CHUNK = 16


def _oddeven_merge(lo, hi, r):
    step = r * 2
    if step < hi - lo:
        yield from _oddeven_merge(lo, hi, step)
        yield from _oddeven_merge(lo + r, hi, step)
        for i in range(lo + r, hi - r, step):
            yield (i, i + r)
    else:
        yield (lo, lo + r)


def _oddeven_merge_sort(lo, hi):
    if (hi - lo) >= 1:
        mid = lo + ((hi - lo) // 2)
        yield from _oddeven_merge_sort(lo, mid)
        yield from _oddeven_merge_sort(mid + 1, hi)
        yield from _oddeven_merge(lo, hi, 1)


def _bitonic_cleanup(n):
    pairs = []
    r = n // 2
    while r >= 1:
        for b in range(0, n, 2 * r):
            for i in range(b, b + r):
                pairs.append((i, i + r))
        r //= 2
    return pairs


_SORT_NET = list(_oddeven_merge_sort(0, _CHUNK - 1))
_CLEAN_NET = _bitonic_cleanup(_LIST)


def _extract_k_smallest(x, k):
    rows, width = x.shape
    lane = jax.lax.broadcasted_iota(jnp.int32, (rows, width), 1)
    mins = []
    for _ in range(k):
        mn = jnp.min(x, axis=1, keepdims=True)
        mins.append(mn)
        cand = jnp.where(x == mn, lane, _BIGI)
        amin = jnp.min(cand, axis=1, keepdims=True)
        x = jnp.where(lane == amin, _INF, x)
    return jnp.concatenate(mins, axis=1)


def _body(q_ref, m_ref, wp_ref, wt_ref, o_ref, cand_ref, *, nb, sub_blocks):
    i = pl.program_id(0)

    @pl.when(i == 0)
    def _init():
        cand_ref[:] = jnp.full(cand_ref.shape, _INF, jnp.bfloat16)

    q = q_ref[:]
    m = m_ref[:]
    hm2 = 0.5 * jnp.sum(m * m, axis=0, keepdims=True)
    qm = jnp.dot(q, m, preferred_element_type=jnp.float32)

    w = 256
    ms = [cand_ref[:, w * j:w * (j + 1)] for j in range(_LIST)]
    for c in range(sub_blocks // _CHUNK):
        b = [(hm2[:, w * s:w * (s + 1)] - qm[:, w * s:w * (s + 1)]
              ).astype(jnp.bfloat16)
             for s in range(c * _CHUNK, (c + 1) * _CHUNK)]
        for x, y in _SORT_NET:
            lo = jnp.minimum(b[x], b[y])
            hi = jnp.maximum(b[x], b[y])
            b[x], b[y] = lo, hi
        for j in range(_LIST):
            ms[j] = jnp.minimum(ms[j], b[_CHUNK - 1 - j])
        for x, y in _CLEAN_NET:
            lo = jnp.minimum(ms[x], ms[y])
            hi = jnp.maximum(ms[x], ms[y])
            ms[x], ms[y] = lo, hi
    for j in range(_LIST):
        cand_ref[:, w * j:w * (j + 1)] = ms[j]

    @pl.when(i == nb - 1)
    def _finalize():
        cand32 = cand_ref[:].astype(jnp.float32)
        top_e = _extract_k_smallest(cand32, K_NEIGHBORS)
        q2 = jnp.sum(q * q, axis=1, keepdims=True)
        nn_d2 = jnp.maximum(2.0 * top_e + q2, 0.0)
        running_mean = jnp.mean(nn_d2) + 1e-12
        kern = KERNEL_EPS / (nn_d2 / running_mean + KERNEL_EPS)
        episodic = 1.0 / jnp.sqrt(jnp.sum(kern, axis=1, keepdims=True) + DENOM_CONST)

        diff = jnp.dot(q, wp_ref[:] - wt_ref[:], preferred_element_type=jnp.float32)
        err = jnp.mean(diff * diff, axis=1, keepdims=True)
        err_mean = jnp.mean(err)
        err_std = jnp.sqrt(jnp.mean((err - err_mean) ** 2)) + 1e-8
        bonus = jnp.clip(1.0 + (err - err_mean) / err_std, 1.0, BONUS_LIMIT)
        o_ref[:] = episodic * bonus


def kernel(queries, memory, w_pred, w_target):
    q_n, d = queries.shape
    m_total = memory.shape[0]
    h = w_pred.shape[1]
    mb = 8192
    nb = pl.cdiv(m_total, mb)
    m_pad = nb * mb
    mem_t = memory.T
    if m_pad != m_total:
        mem_t = jnp.pad(mem_t, ((0, 0), (0, m_pad - m_total)),
                        constant_values=_PAD_VALUE)

    out = pl.pallas_call(
        functools.partial(_body, nb=nb, sub_blocks=mb // 256),
        grid=(nb,),
        in_specs=[
            pl.BlockSpec((q_n, d), lambda i: (0, 0)),
            pl.BlockSpec((d, mb), lambda i: (0, i)),
            pl.BlockSpec((d, h), lambda i: (0, 0)),
            pl.BlockSpec((d, h), lambda i: (0, 0)),
        ],
        out_specs=pl.BlockSpec((q_n, 1), lambda i: (0, 0)),
        out_shape=jax.ShapeDtypeStruct((q_n, 1), jnp.float32),
        scratch_shapes=[pltpu.VMEM((q_n, 256 * _LIST), jnp.bfloat16)],
    )(queries, mem_t, w_pred, w_target)
    return out[:, 0]


# speedup vs baseline: 12.9967x; 1.0298x over previous
import functools

import jax
import jax.numpy as jnp
from jax.experimental import pallas as pl
from jax.experimental.pallas import tpu as pltpu

K_NEIGHBORS = 10
KERNEL_EPS = 1e-5
DENOM_CONST = 1e-5
BONUS_LIMIT = 5.0

_INF = float("inf")
_BIGI = 2**30
_PAD_VALUE = 1.0e6
_LIST = 16
_CHUNK = 16


def _oddeven_merge(lo, hi, r):
    step = r * 2
    if step < hi - lo:
        yield from _oddeven_merge(lo, hi, step)
        yield from _oddeven_merge(lo + r, hi, step)
        for i in range(lo + r, hi - r, step):
            yield (i, i + r)
    else:
        yield (lo, lo + r)


def _oddeven_merge_sort(lo, hi):
    if (hi - lo) >= 1:
        mid = lo + ((hi - lo) // 2)
        yield from _oddeven_merge_sort(lo, mid)
        yield from _oddeven_merge_sort(mid + 1, hi)
        yield from _oddeven_merge(lo, hi, 1)


def _bitonic_cleanup(n):
    pairs = []
    r = n // 2
    while r >= 1:
        for b in range(0, n, 2 * r):
            for i in range(b, b + r):
                pairs.append((i, i + r))
        r //= 2
    return pairs


_SORT_NET = list(_oddeven_merge_sort(0, _CHUNK - 1))
_CLEAN_NET = _bitonic_cleanup(_LIST)


def _extract_k_largest(x, k):
    rows, width = x.shape
    lane = jax.lax.broadcasted_iota(jnp.int32, (rows, width), 1)
    maxs = []
    for _ in range(k):
        mx = jnp.max(x, axis=1, keepdims=True)
        maxs.append(mx)
        cand = jnp.where(x == mx, lane, _BIGI)
        amax = jnp.min(cand, axis=1, keepdims=True)
        x = jnp.where(lane == amax, -_INF, x)
    return jnp.concatenate(maxs, axis=1)


def _body(q_ref, m_ref, wp_ref, wt_ref, o_ref, cand_ref, *, nb, sub_blocks):
    i = pl.program_id(0)

    @pl.when(i == 0)
    def _init():
        cand_ref[:] = jnp.full(cand_ref.shape, -_INF, jnp.bfloat16)

    q = q_ref[:]
    m = m_ref[:]
    hm2 = 0.5 * jnp.sum(m * m, axis=0, keepdims=True)
    q_aug = jnp.concatenate(
        [q, jnp.ones((q.shape[0], 1), jnp.float32)], axis=1)
    m_aug = jnp.concatenate([m, -hm2], axis=0)
    ne = jnp.dot(q_aug, m_aug, preferred_element_type=jnp.float32)

    w = 256
    ms = [cand_ref[:, w * j:w * (j + 1)] for j in range(_LIST)]
    for c in range(sub_blocks // _CHUNK):
        b = [ne[:, w * s:w * (s + 1)].astype(jnp.bfloat16)
             for s in range(c * _CHUNK, (c + 1) * _CHUNK)]
        for x, y in _SORT_NET:
            lo = jnp.minimum(b[x], b[y])
            hi = jnp.maximum(b[x], b[y])
            b[x], b[y] = lo, hi
        for j in range(_LIST):
            ms[j] = jnp.maximum(ms[j], b[_CHUNK - 1 - j])
        for x, y in _CLEAN_NET:
            lo = jnp.minimum(ms[x], ms[y])
            hi = jnp.maximum(ms[x], ms[y])
            ms[x], ms[y] = lo, hi
    for j in range(_LIST):
        cand_ref[:, w * j:w * (j + 1)] = ms[j]

    @pl.when(i == nb - 1)
    def _finalize():
        cand32 = cand_ref[:].astype(jnp.float32)
        top_ne = _extract_k_largest(cand32, K_NEIGHBORS)
        q2 = jnp.sum(q * q, axis=1, keepdims=True)
        nn_d2 = jnp.maximum(q2 - 2.0 * top_ne, 0.0)
        running_mean = jnp.mean(nn_d2) + 1e-12
        kern = KERNEL_EPS / (nn_d2 / running_mean + KERNEL_EPS)
        episodic = 1.0 / jnp.sqrt(jnp.sum(kern, axis=1, keepdims=True) + DENOM_CONST)

        diff = jnp.dot(q, wp_ref[:] - wt_ref[:], preferred_element_type=jnp.float32)
        err = jnp.mean(diff * diff, axis=1, keepdims=True)
        err_mean = jnp.mean(err)
        err_std = jnp.sqrt(jnp.mean((err - err_mean) ** 2)) + 1e-8
        bonus = jnp.clip(1.0 + (err - err_mean) / err_std, 1.0, BONUS_LIMIT)
        o_ref[:] = episodic * bonus


def kernel(queries, memory, w_pred, w_target):
    q_n, d = queries.shape
    m_total = memory.shape[0]
    h = w_pred.shape[1]
    mb = 8192
    nb = pl.cdiv(m_total, mb)
    m_pad = nb * mb
    mem_t = memory.T
    if m_pad != m_total:
        mem_t = jnp.pad(mem_t, ((0, 0), (0, m_pad - m_total)),
                        constant_values=_PAD_VALUE)

    out = pl.pallas_call(
        functools.partial(_body, nb=nb, sub_blocks=mb // 256),
        grid=(nb,),
        in_specs=[
            pl.BlockSpec((q_n, d), lambda i: (0, 0)),
            pl.BlockSpec((d, mb), lambda i: (0, i)),
            pl.BlockSpec((d, h), lambda i: (0, 0)),
            pl.BlockSpec((d, h), lambda i: (0, 0)),
        ],
        out_specs=pl.BlockSpec((q_n, 1), lambda i: (0, 0)),
        out_shape=jax.ShapeDtypeStruct((q_n, 1), jnp.float32),
        scratch_shapes=[pltpu.VMEM((q_n, 256 * _LIST), jnp.bfloat16)],
    )(queries, mem_t, w_pred, w_target)
    return out[:, 0]
